# Initial kernel scaffold; baseline (speedup 1.0000x reference)
import jax
import jax.numpy as jnp
from jax.experimental import pallas as pl


def kernel(x, W, b):
    raise NotImplementedError("write your pallas kernel here")


# speedup vs baseline: 1.3549x; 1.3549x over previous
import functools

import jax
import jax.numpy as jnp
from jax.experimental import pallas as pl

_TOK_BLOCK = 512


def _router_kernel(x_ref, w_ref, b_ref, probs_ref, idx_ref, wts_ref):
    x = x_ref[...]
    logits = jax.lax.dot_general(
        x, w_ref[...], (((1,), (1,)), ((), ())),
        preferred_element_type=jnp.float32,
    )
    logits = logits + b_ref[...]

    m = jnp.max(logits, axis=1, keepdims=True)
    e = jnp.exp(logits - m)
    z = jnp.sum(e, axis=1, keepdims=True)
    probs = e / z
    probs_ref[...] = probs

    cols = jax.lax.broadcasted_iota(jnp.int32, probs.shape, 1)
    v1 = jnp.max(probs, axis=1, keepdims=True)
    i1 = jnp.min(jnp.where(probs == v1, cols, probs.shape[1]), axis=1,
                 keepdims=True)
    masked = jnp.where(cols == i1, -jnp.inf, probs)
    v2 = jnp.max(masked, axis=1, keepdims=True)
    i2 = jnp.min(jnp.where(masked == v2, cols, probs.shape[1]), axis=1,
                 keepdims=True)

    denom = jnp.maximum(v1 + v2, 1e-9)
    wts_ref[...] = jnp.concatenate([v1 / denom, v2 / denom], axis=1)
    idx_ref[...] = jnp.concatenate([i1, i2], axis=1)


@jax.jit
def kernel(x, W, b):
    n_tok, d_model = x.shape
    n_exp = W.shape[0]
    t = _TOK_BLOCK
    grid = (n_tok // t,)
    probs, idx, wts = pl.pallas_call(
        _router_kernel,
        grid=grid,
        in_specs=[
            pl.BlockSpec((t, d_model), lambda i: (i, 0)),
            pl.BlockSpec((n_exp, d_model), lambda i: (0, 0)),
            pl.BlockSpec((1, n_exp), lambda i: (0, 0)),
        ],
        out_specs=[
            pl.BlockSpec((t, n_exp), lambda i: (i, 0)),
            pl.BlockSpec((t, 2), lambda i: (i, 0)),
            pl.BlockSpec((t, 2), lambda i: (i, 0)),
        ],
        out_shape=[
            jax.ShapeDtypeStruct((n_tok, n_exp), jnp.float32),
            jax.ShapeDtypeStruct((n_tok, 2), jnp.int32),
            jax.ShapeDtypeStruct((n_tok, 2), jnp.float32),
        ],
    )(x, W.reshape(n_exp, d_model), b.reshape(1, n_exp))
    return probs, idx, wts


# speedup vs baseline: 1.5620x; 1.1528x over previous
import functools

import jax
import jax.numpy as jnp
from jax.experimental import pallas as pl

_TOK_BLOCK = 1024


def _router_kernel(x_ref, w_ref, b_ref, probs_ref, idx_ref, wts_ref):
    x = x_ref[...]
    logits = jax.lax.dot_general(
        x, w_ref[...], (((1,), (1,)), ((), ())),
        preferred_element_type=jnp.float32,
    )
    logits = logits + b_ref[...]

    m = jnp.max(logits, axis=1, keepdims=True)
    e = jnp.exp(logits - m)
    z = jnp.sum(e, axis=1, keepdims=True)
    probs = e / z
    probs_ref[...] = probs

    cols = jax.lax.broadcasted_iota(jnp.int32, probs.shape, 1)
    v1 = jnp.max(probs, axis=1, keepdims=True)
    i1 = jnp.min(jnp.where(probs == v1, cols, probs.shape[1]), axis=1,
                 keepdims=True)
    masked = jnp.where(cols == i1, -jnp.inf, probs)
    v2 = jnp.max(masked, axis=1, keepdims=True)
    i2 = jnp.min(jnp.where(masked == v2, cols, probs.shape[1]), axis=1,
                 keepdims=True)

    denom = jnp.maximum(v1 + v2, 1e-9)
    wts_ref[...] = jnp.concatenate([v1 / denom, v2 / denom], axis=1)
    idx_ref[...] = jnp.concatenate([i1, i2], axis=1)


@jax.jit
def kernel(x, W, b):
    n_tok, d_model = x.shape
    n_exp = W.shape[0]
    t = _TOK_BLOCK
    grid = (n_tok // t,)
    probs, idx, wts = pl.pallas_call(
        _router_kernel,
        grid=grid,
        in_specs=[
            pl.BlockSpec((t, d_model), lambda i: (i, 0)),
            pl.BlockSpec((n_exp, d_model), lambda i: (0, 0)),
            pl.BlockSpec((1, n_exp), lambda i: (0, 0)),
        ],
        out_specs=[
            pl.BlockSpec((t, n_exp), lambda i: (i, 0)),
            pl.BlockSpec((t, 2), lambda i: (i, 0)),
            pl.BlockSpec((t, 2), lambda i: (i, 0)),
        ],
        out_shape=[
            jax.ShapeDtypeStruct((n_tok, n_exp), jnp.float32),
            jax.ShapeDtypeStruct((n_tok, 2), jnp.int32),
            jax.ShapeDtypeStruct((n_tok, 2), jnp.float32),
        ],
    )(x, W.reshape(n_exp, d_model), b.reshape(1, n_exp))
    return probs, idx, wts


# speedup vs baseline: 1.6192x; 1.0366x over previous
import functools

import jax
import jax.numpy as jnp
from jax.experimental import pallas as pl

_TOK_BLOCK = 2048


def _router_kernel(x_ref, w_ref, b_ref, probs_ref, idx_ref, wts_ref):
    x = x_ref[...]
    logits = jax.lax.dot_general(
        x, w_ref[...], (((1,), (1,)), ((), ())),
        preferred_element_type=jnp.float32,
    )
    logits = logits + b_ref[...]

    m = jnp.max(logits, axis=1, keepdims=True)
    e = jnp.exp(logits - m)
    z = jnp.sum(e, axis=1, keepdims=True)
    probs = e / z
    probs_ref[...] = probs

    cols = jax.lax.broadcasted_iota(jnp.int32, probs.shape, 1)
    v1 = jnp.max(probs, axis=1, keepdims=True)
    i1 = jnp.min(jnp.where(probs == v1, cols, probs.shape[1]), axis=1,
                 keepdims=True)
    masked = jnp.where(cols == i1, -jnp.inf, probs)
    v2 = jnp.max(masked, axis=1, keepdims=True)
    i2 = jnp.min(jnp.where(masked == v2, cols, probs.shape[1]), axis=1,
                 keepdims=True)

    denom = jnp.maximum(v1 + v2, 1e-9)
    wts_ref[...] = jnp.concatenate([v1 / denom, v2 / denom], axis=1)
    idx_ref[...] = jnp.concatenate([i1, i2], axis=1)


@jax.jit
def kernel(x, W, b):
    n_tok, d_model = x.shape
    n_exp = W.shape[0]
    t = _TOK_BLOCK
    grid = (n_tok // t,)
    probs, idx, wts = pl.pallas_call(
        _router_kernel,
        grid=grid,
        in_specs=[
            pl.BlockSpec((t, d_model), lambda i: (i, 0)),
            pl.BlockSpec((n_exp, d_model), lambda i: (0, 0)),
            pl.BlockSpec((1, n_exp), lambda i: (0, 0)),
        ],
        out_specs=[
            pl.BlockSpec((t, n_exp), lambda i: (i, 0)),
            pl.BlockSpec((t, 2), lambda i: (i, 0)),
            pl.BlockSpec((t, 2), lambda i: (i, 0)),
        ],
        out_shape=[
            jax.ShapeDtypeStruct((n_tok, n_exp), jnp.float32),
            jax.ShapeDtypeStruct((n_tok, 2), jnp.int32),
            jax.ShapeDtypeStruct((n_tok, 2), jnp.float32),
        ],
    )(x, W.reshape(n_exp, d_model), b.reshape(1, n_exp))
    return probs, idx, wts
